# baseline (speedup 1.0000x reference)
import functools

import jax
import jax.numpy as jnp
from jax import lax
from jax.experimental import pallas as pl
from jax.experimental.pallas import tpu as pltpu
from jax.experimental.pallas import tpu_sc as plsc

N = 10000
D = 128
HC = D // 2
NP = 10240
NC = 2
NS = 16
NW = NC * NS
K = 256
E = 320000
CPT = 81
EPAD = NS * CPT * K + 2 * K
KG = 192
CPTG = 108
KA = 128
CPTA = 162
KD = 128
CPD = (NS * CPT * K) // (NW * KD)
RPT = NP // NS


def _zero_rows(rows_v):
    z16 = jnp.zeros((16,), jnp.float32)

    def zrow(i, _):
        for j in range(HC // 16):
            rows_v[i, pl.ds(j * 16, 16)] = z16
        return 0
    lax.fori_loop(0, rows_v.shape[0], zrow, 0)


def _zero_vec(den_v):
    z16 = jnp.zeros((16,), jnp.float32)

    def zden(i, _):
        den_v[pl.ds(pl.multiple_of(i * 16, 16), 16)] = z16
        return 0
    lax.fori_loop(0, NP // 16, zden, 0)


def _stage_h(hv_hbm, h_sh, sid):
    base = sid * RPT
    pltpu.sync_copy(hv_hbm.at[pl.ds(base, RPT)], h_sh.at[pl.ds(base, RPT)])


def _zero_num_shared(rows_v, num_sh, sid):
    base = sid * RPT
    k = rows_v.shape[0]
    for r in range(RPT // k):
        pltpu.sync_copy(rows_v, num_sh.at[pl.ds(base + r * k, k)])
    rem = RPT % k
    if rem:
        pltpu.sync_copy(rows_v.at[pl.ds(0, rem)],
                        num_sh.at[pl.ds(base + (RPT // k) * k, rem)])


def _edge_pipeline(src_h, dst_h, hv, num_h, num_sh, srcs, dsts, rows,
                   gsems, ssems, compute_fn, cid, sid, k, cpt):
    reg = sid * cpt

    def load_and_gather(c, b):
        base = (reg + c) * k
        pltpu.sync_copy(src_h.at[pl.ds(base, k)], srcs[b])
        pltpu.sync_copy(dst_h.at[pl.ds(base, k)], dsts[b])
        pltpu.async_copy(hv.at[srcs[b]], rows[b], gsems[b])

    load_and_gather(0, 0)
    load_and_gather(1, 1)

    def outer(cc, _):
        for b in range(3):
            c = cc * 3 + b
            bp = (b + 2) % 3
            pltpu.make_async_copy(hv.at[srcs[b]], rows[b], gsems[b]).wait()
            if compute_fn is not None:
                compute_fn(srcs[b], dsts[b], rows[b])
            pltpu.async_copy(rows[b], num_sh.at[dsts[b]], ssems[b], add=True)

            @pl.when(c >= 1)
            def _():
                pltpu.make_async_copy(
                    rows[bp], num_sh.at[dsts[bp]], ssems[bp]).wait()
            load_and_gather(c + 2, bp)
        return 0
    lax.fori_loop(0, cpt // 3, outer, 0)

    pltpu.make_async_copy(hv.at[srcs[0]], rows[0], gsems[0]).wait()
    pltpu.make_async_copy(hv.at[srcs[1]], rows[1], gsems[1]).wait()
    lb = (cpt - 1) % 3
    pltpu.make_async_copy(rows[lb], num_sh.at[dsts[lb]], ssems[lb]).wait()

    plsc.subcore_barrier()
    pltpu.sync_copy(num_sh.at[pl.ds(sid * RPT, RPT)],
                    num_h.at[cid, pl.ds(sid * RPT, RPT)])


def _gat_edges_body(src_h, dst_h, h_h, asrc_h, adst_h, num_h, den_h,
                    num_sh, h_sh, asrc_sh, adst_sh, den_v, es_v, ed_v,
                    src0, src1, src2, dst0, dst1, dst2,
                    rows0, rows1, rows2, gs0, gs1, gs2, ss0, ss1, ss2):
    cid = lax.axis_index("c")
    sid = lax.axis_index("s")
    base = sid * RPT
    _stage_h(h_h.at[cid], h_sh, sid)
    pltpu.sync_copy(asrc_h.at[pl.ds(base, RPT)], asrc_sh.at[pl.ds(base, RPT)])
    pltpu.sync_copy(adst_h.at[pl.ds(base, RPT)], adst_sh.at[pl.ds(base, RPT)])
    _zero_rows(rows0)
    _zero_num_shared(rows0, num_sh, sid)
    _zero_vec(den_v)
    plsc.subcore_barrier()

    def compute(src_v, dst_v, rows_v):
        pltpu.sync_copy(asrc_sh.at[src_v], es_v)
        pltpu.sync_copy(adst_sh.at[dst_v], ed_v)

        def group(g, _):
            gbase = pl.multiple_of(g * 16, 16)
            d16 = dst_v[pl.ds(gbase, 16)]
            e = es_v[pl.ds(gbase, 16)] + ed_v[pl.ds(gbase, 16)]
            e = jnp.where(e >= 0.0, e, 0.2 * e)
            w = jnp.exp(e)
            plsc.addupdate_scatter(den_v, [d16], w)
            for l in range(16):
                ws = w[l]
                row = g * 16 + l
                for j in range(HC // 16):
                    sl = pl.ds(j * 16, 16)
                    rows_v[row, sl] = rows_v[row, sl] * ws
            return 0
        lax.fori_loop(0, KA // 16, group, 0)

    _edge_pipeline(src_h, dst_h, h_sh, num_h, num_sh,
                   (src0, src1, src2), (dst0, dst1, dst2),
                   (rows0, rows1, rows2), (gs0, gs1, gs2), (ss0, ss1, ss2),
                   compute, cid, sid, KA, CPTA)

    @pl.when(cid == 0)
    def _():
        pltpu.sync_copy(den_v, den_h.at[sid])


def _gcn_edges_body(src_h, dst_h, h_h, num_h,
                    num_sh, h_sh, src0, src1, src2, dst0, dst1, dst2,
                    rows0, rows1, rows2, gs0, gs1, gs2, ss0, ss1, ss2):
    cid = lax.axis_index("c")
    sid = lax.axis_index("s")
    _stage_h(h_h.at[cid], h_sh, sid)
    _zero_rows(rows0)
    _zero_num_shared(rows0, num_sh, sid)
    plsc.subcore_barrier()
    _edge_pipeline(src_h, dst_h, h_sh, num_h, num_sh,
                   (src0, src1, src2), (dst0, dst1, dst2),
                   (rows0, rows1, rows2), (gs0, gs1, gs2), (ss0, ss1, ss2),
                   None, cid, sid, KG, CPTG)


def _degrees_body(dst_h, deg_h, den_v, dst_v):
    cid = lax.axis_index("c")
    sid = lax.axis_index("s")
    gw = cid * NS + sid
    _zero_vec(den_v)
    ones = jnp.ones((16,), jnp.float32)

    def chunk(c, _):
        base = (gw * CPD + c) * KD
        pltpu.sync_copy(dst_h.at[pl.ds(base, KD)], dst_v)
        for g in range(KD // 16):
            d16 = dst_v[pl.ds(g * 16, 16)]
            plsc.addupdate_scatter(den_v, [d16], ones)
        return 0
    lax.fori_loop(0, CPD, chunk, 0)
    pltpu.sync_copy(den_v, deg_h.at[gw])


@functools.lru_cache(maxsize=None)
def _sc_kernels():
    mesh = plsc.VectorSubcoreMesh(core_axis_name="c", subcore_axis_name="s",
                                  num_cores=NC, num_subcores=NS)
    cparams = pltpu.CompilerParams(needs_layout_passes=False,
                                   use_tc_tiling_on_sc=False)
    idx6a = [pltpu.VMEM((KA,), jnp.int32)] * 6
    rows3a = [pltpu.VMEM((KA, HC), jnp.float32)] * 3
    idx6g = [pltpu.VMEM((KG,), jnp.int32)] * 6
    rows3g = [pltpu.VMEM((KG, HC), jnp.float32)] * 3
    sems6 = [pltpu.SemaphoreType.DMA] * 6
    gat = pl.kernel(
        _gat_edges_body,
        out_type=(jax.ShapeDtypeStruct((NC, NP, HC), jnp.float32),
                  jax.ShapeDtypeStruct((NS, NP), jnp.float32)),
        mesh=mesh,
        compiler_params=cparams,
        scratch_types=[
            pltpu.VMEM_SHARED((NP, HC), jnp.float32),
            pltpu.VMEM_SHARED((NP, HC), jnp.float32),
            pltpu.VMEM_SHARED((NP,), jnp.float32),
            pltpu.VMEM_SHARED((NP,), jnp.float32),
            pltpu.VMEM((NP,), jnp.float32),
            pltpu.VMEM((KA,), jnp.float32),
            pltpu.VMEM((KA,), jnp.float32),
        ] + idx6a + rows3a + sems6,
    )
    gcn = pl.kernel(
        _gcn_edges_body,
        out_type=jax.ShapeDtypeStruct((NC, NP, HC), jnp.float32),
        mesh=mesh,
        compiler_params=cparams,
        scratch_types=[
            pltpu.VMEM_SHARED((NP, HC), jnp.float32),
            pltpu.VMEM_SHARED((NP, HC), jnp.float32),
        ] + idx6g + rows3g + sems6,
    )
    deg = pl.kernel(
        _degrees_body,
        out_type=jax.ShapeDtypeStruct((NW, NP), jnp.float32),
        mesh=mesh,
        compiler_params=cparams,
        scratch_types=[
            pltpu.VMEM((NP,), jnp.float32),
            pltpu.VMEM((KD,), jnp.int32),
        ],
    )
    return gat, gcn, deg


_BM = 256


def _split(h):
    return h[:, :HC], h[:, HC:]


def _mm(x, W):
    def body(x_ref, w_ref, o_ref):
        h = jnp.dot(x_ref[...], w_ref[...], preferred_element_type=jnp.float32)
        lo, hi = _split(h)
        o_ref[0] = lo
        o_ref[1] = hi
    return pl.pallas_call(
        body,
        grid=(NP // _BM,),
        in_specs=[pl.BlockSpec((_BM, D), lambda i: (i, 0)),
                  pl.BlockSpec((D, D), lambda i: (0, 0))],
        out_specs=pl.BlockSpec((NC, _BM, HC), lambda i: (0, i, 0)),
        out_shape=jax.ShapeDtypeStruct((NC, NP, HC), jnp.float32),
    )(x, W)


def _mm_scale(x, W, dinv):
    def body(x_ref, w_ref, di_ref, o_ref):
        h = di_ref[...] * jnp.dot(x_ref[...], w_ref[...],
                                  preferred_element_type=jnp.float32)
        lo, hi = _split(h)
        o_ref[0] = lo
        o_ref[1] = hi
    return pl.pallas_call(
        body,
        grid=(NP // _BM,),
        in_specs=[pl.BlockSpec((_BM, D), lambda i: (i, 0)),
                  pl.BlockSpec((D, D), lambda i: (0, 0)),
                  pl.BlockSpec((_BM, 1), lambda i: (i, 0))],
        out_specs=pl.BlockSpec((NC, _BM, HC), lambda i: (0, i, 0)),
        out_shape=jax.ShapeDtypeStruct((NC, NP, HC), jnp.float32),
    )(x, W, dinv)


def _mm_att(x, W, a2):
    def body(x_ref, w_ref, a_ref, h_ref, as_ref, ad_ref):
        h = jnp.dot(x_ref[...], w_ref[...], preferred_element_type=jnp.float32)
        av = jnp.dot(h, a_ref[...], preferred_element_type=jnp.float32)
        lo, hi = _split(h)
        h_ref[0] = lo
        h_ref[1] = hi
        as_ref[...] = av[:, 0:1]
        ad_ref[...] = av[:, 1:2]
    return pl.pallas_call(
        body,
        grid=(NP // _BM,),
        in_specs=[pl.BlockSpec((_BM, D), lambda i: (i, 0)),
                  pl.BlockSpec((D, D), lambda i: (0, 0)),
                  pl.BlockSpec((D, 2), lambda i: (0, 0))],
        out_specs=[pl.BlockSpec((NC, _BM, HC), lambda i: (0, i, 0)),
                   pl.BlockSpec((_BM, 1), lambda i: (i, 0)),
                   pl.BlockSpec((_BM, 1), lambda i: (i, 0))],
        out_shape=[jax.ShapeDtypeStruct((NC, NP, HC), jnp.float32),
                   jax.ShapeDtypeStruct((NP, 1), jnp.float32),
                   jax.ShapeDtypeStruct((NP, 1), jnp.float32)],
    )(x, W, a2)


def _cat(ref):
    return jnp.concatenate([ref[0], ref[1]], axis=-1)


def _gat_epilogue(num, den, h, asrc, adst, b, relu):
    def body(n_ref, d_ref, h_ref, as_ref, ad_ref, b_ref, o_ref):
        e = as_ref[...] + ad_ref[...]
        w_self = jnp.exp(jnp.where(e >= 0.0, e, 0.2 * e))
        den_tot = jnp.sum(d_ref[...], axis=0, keepdims=True).T + w_self + 1e-16
        numer = _cat(n_ref) + w_self * _cat(h_ref)
        out = numer / den_tot + b_ref[...]
        if relu:
            out = jnp.maximum(out, 0.0)
        o_ref[...] = out
    return pl.pallas_call(
        body,
        grid=(NP // _BM,),
        in_specs=[pl.BlockSpec((NC, _BM, HC), lambda i: (0, i, 0)),
                  pl.BlockSpec((NS, _BM), lambda i: (0, i)),
                  pl.BlockSpec((NC, _BM, HC), lambda i: (0, i, 0)),
                  pl.BlockSpec((_BM, 1), lambda i: (i, 0)),
                  pl.BlockSpec((_BM, 1), lambda i: (i, 0)),
                  pl.BlockSpec((1, D), lambda i: (0, 0))],
        out_specs=pl.BlockSpec((_BM, D), lambda i: (i, 0)),
        out_shape=jax.ShapeDtypeStruct((NP, D), jnp.float32),
    )(num, den, h, asrc, adst, b.reshape(1, D))


def _gcn_epilogue(num, hp, dinv, b, relu):
    def body(n_ref, h_ref, di_ref, b_ref, o_ref):
        out = di_ref[...] * (_cat(n_ref) + _cat(h_ref)) + b_ref[...]
        if relu:
            out = jnp.maximum(out, 0.0)
        o_ref[...] = out
    return pl.pallas_call(
        body,
        grid=(NP // _BM,),
        in_specs=[pl.BlockSpec((NC, _BM, HC), lambda i: (0, i, 0)),
                  pl.BlockSpec((NC, _BM, HC), lambda i: (0, i, 0)),
                  pl.BlockSpec((_BM, 1), lambda i: (i, 0)),
                  pl.BlockSpec((1, D), lambda i: (0, 0))],
        out_specs=pl.BlockSpec((_BM, D), lambda i: (i, 0)),
        out_shape=jax.ShapeDtypeStruct((NP, D), jnp.float32),
    )(num, hp, dinv, b.reshape(1, D))


def _dinv_kernel(deg):
    def body(deg_ref, o_ref):
        tot = jnp.sum(deg_ref[...], axis=0, keepdims=True) + 1.0
        o_ref[...] = lax.rsqrt(tot)
    return pl.pallas_call(
        body,
        out_shape=jax.ShapeDtypeStruct((1, NP), jnp.float32),
    )(deg)


def _fusion(e1, e2, c1, c2, aW1, ab1, aW2):
    def body(e1_ref, e2_ref, c1_ref, c2_ref, w1_ref, b1_ref, w2_ref, o_ref):
        z0 = e1_ref[...]
        z1 = e2_ref[...]
        z2 = (c1_ref[...] + c2_ref[...]) * 0.5

        def att(z):
            t = jnp.tanh(jnp.dot(z, w1_ref[...],
                                 preferred_element_type=jnp.float32)
                         + b1_ref[...])
            return jnp.dot(t, w2_ref[...], preferred_element_type=jnp.float32)

        w0, w1, w2 = att(z0), att(z1), att(z2)
        m = jnp.maximum(jnp.maximum(w0, w1), w2)
        x0 = jnp.exp(w0 - m)
        x1 = jnp.exp(w1 - m)
        x2 = jnp.exp(w2 - m)
        s = x0 + x1 + x2
        o_ref[...] = (x0 * z0 + x1 * z1 + x2 * z2) / s
    return pl.pallas_call(
        body,
        grid=(NP // _BM,),
        in_specs=[pl.BlockSpec((_BM, D), lambda i: (i, 0))] * 4
                 + [pl.BlockSpec((D, 16), lambda i: (0, 0)),
                    pl.BlockSpec((1, 16), lambda i: (0, 0)),
                    pl.BlockSpec((16, 1), lambda i: (0, 0))],
        out_specs=pl.BlockSpec((_BM, D), lambda i: (i, 0)),
        out_shape=jax.ShapeDtypeStruct((NP, D), jnp.float32),
    )(e1, e2, c1, c2, aW1, ab1, aW2)


def _gat_layer(x, W, a_s, a_d, b, src, dst, relu):
    gat_edges, _, _ = _sc_kernels()
    h, asrc, adst = _mm_att(x, W, jnp.stack([a_s, a_d], axis=1))
    num, den = gat_edges(src, dst, h, asrc.reshape(NP), adst.reshape(NP))
    return _gat_epilogue(num, den, h, asrc, adst, b, relu)


def _gcn_layer(x, W, b, src, dst, dinv_col, relu):
    _, gcn_edges, _ = _sc_kernels()
    hp = _mm_scale(x, W, dinv_col)
    num = gcn_edges(src, dst, hp)
    return _gcn_epilogue(num, hp, dinv_col, b, relu)


def kernel(adj, aug_feat1, aug_feat2,
           g1_W1, g1_as1, g1_ad1, g1_b1, g1_W2, g1_as2, g1_ad2, g1_b2,
           g2_W1, g2_as1, g2_ad1, g2_b1, g2_W2, g2_as2, g2_ad2, g2_b2,
           c_W1, c_b1, c_W2, c_b2, a_W1, a_b1, a_W2):
    pad_e = jnp.full((EPAD - E,), N, jnp.int32)
    src = jnp.concatenate([adj[0], pad_e])
    dst = jnp.concatenate([adj[1], pad_e])
    x1 = jnp.pad(aug_feat1, ((0, NP - N), (0, 0)))
    x2 = jnp.pad(aug_feat2, ((0, NP - N), (0, 0)))

    _, _, degrees = _sc_kernels()
    deg = degrees(dst)
    dinv_col = _dinv_kernel(deg).reshape(NP, 1)

    h1 = _gat_layer(x1, g1_W1, g1_as1, g1_ad1, g1_b1, src, dst, True)
    emb1 = _gat_layer(h1, g1_W2, g1_as2, g1_ad2, g1_b2, src, dst, False)
    h2 = _gat_layer(x2, g2_W1, g2_as1, g2_ad1, g2_b1, src, dst, True)
    emb2 = _gat_layer(h2, g2_W2, g2_as2, g2_ad2, g2_b2, src, dst, False)

    hc1 = _gcn_layer(x1, c_W1, c_b1, src, dst, dinv_col, True)
    com1 = _gcn_layer(hc1, c_W2, c_b2, src, dst, dinv_col, False)
    hc2 = _gcn_layer(x2, c_W1, c_b1, src, dst, dinv_col, True)
    com2 = _gcn_layer(hc2, c_W2, c_b2, src, dst, dinv_col, False)

    out = _fusion(emb1, emb2, com1, com2, a_W1, a_b1.reshape(1, 16), a_W2)
    return out[:N]


# speedup vs baseline: 1.0900x; 1.0900x over previous
import functools

import jax
import jax.numpy as jnp
from jax import lax
from jax.experimental import pallas as pl
from jax.experimental.pallas import tpu as pltpu
from jax.experimental.pallas import tpu_sc as plsc

N = 10000
D = 128
HC = D // 2
NP = 10240
NC = 2
NS = 16
NW = NC * NS
K = 256
E = 320000
CPT = 81
EPAD = NS * CPT * K + 2 * K
KG = 192
CPTG = 108
KA = 128
CPTA = 162
KD = 128
CPD = (NS * CPT * K) // (NW * KD)
RPT = NP // NS


def _zero_rows(rows_v):
    z16 = jnp.zeros((16,), jnp.float32)

    def zrow(i, _):
        for j in range(HC // 16):
            rows_v[i, pl.ds(j * 16, 16)] = z16
        return 0
    lax.fori_loop(0, rows_v.shape[0], zrow, 0)


def _zero_vec(den_v):
    z16 = jnp.zeros((16,), jnp.float32)

    def zden(i, _):
        den_v[pl.ds(pl.multiple_of(i * 16, 16), 16)] = z16
        return 0
    lax.fori_loop(0, NP // 16, zden, 0)


def _stage_h(hv_hbm, h_sh, sid):
    base = sid * RPT
    pltpu.sync_copy(hv_hbm.at[pl.ds(base, RPT)], h_sh.at[pl.ds(base, RPT)])


def _zero_num_shared(rows_v, num_sh, sid):
    base = sid * RPT
    k = rows_v.shape[0]
    for r in range(RPT // k):
        pltpu.sync_copy(rows_v, num_sh.at[pl.ds(base + r * k, k)])
    rem = RPT % k
    if rem:
        pltpu.sync_copy(rows_v.at[pl.ds(0, rem)],
                        num_sh.at[pl.ds(base + (RPT // k) * k, rem)])


def _edge_pipeline(src_h, dst_h, hv, num_h, num_sh, srcs, dsts, rows,
                   gsems, ssems, compute_fn, cid, sid, k, cpt,
                   extra_gather_fn=None, extra_drain_fn=None):
    reg = sid * cpt

    def load_and_gather(c, b):
        base = (reg + c) * k
        pltpu.sync_copy(src_h.at[pl.ds(base, k)], srcs[b])
        pltpu.sync_copy(dst_h.at[pl.ds(base, k)], dsts[b])
        pltpu.async_copy(hv.at[srcs[b]], rows[b], gsems[b])
        if extra_gather_fn is not None:
            extra_gather_fn(b)

    load_and_gather(0, 0)
    load_and_gather(1, 1)

    def outer(cc, _):
        for b in range(3):
            c = cc * 3 + b
            bp = (b + 2) % 3
            pltpu.make_async_copy(hv.at[srcs[b]], rows[b], gsems[b]).wait()
            if compute_fn is not None:
                compute_fn(srcs[b], dsts[b], rows[b], b)
            pltpu.async_copy(rows[b], num_sh.at[dsts[b]], ssems[b], add=True)

            @pl.when(c >= 1)
            def _():
                pltpu.make_async_copy(
                    rows[bp], num_sh.at[dsts[bp]], ssems[bp]).wait()
            load_and_gather(c + 2, bp)
        return 0
    lax.fori_loop(0, cpt // 3, outer, 0)

    pltpu.make_async_copy(hv.at[srcs[0]], rows[0], gsems[0]).wait()
    pltpu.make_async_copy(hv.at[srcs[1]], rows[1], gsems[1]).wait()
    if extra_drain_fn is not None:
        extra_drain_fn(0)
        extra_drain_fn(1)
    lb = (cpt - 1) % 3
    pltpu.make_async_copy(rows[lb], num_sh.at[dsts[lb]], ssems[lb]).wait()

    plsc.subcore_barrier()
    pltpu.sync_copy(num_sh.at[pl.ds(sid * RPT, RPT)],
                    num_h.at[cid, pl.ds(sid * RPT, RPT)])


def _gat_edges_body(src_h, dst_h, h_h, asrc_h, adst_h, num_h, den_h,
                    num_sh, h_sh, asrc_sh, adst_sh, den_v,
                    es0, es1, es2, ed0, ed1, ed2,
                    src0, src1, src2, dst0, dst1, dst2,
                    rows0, rows1, rows2, gs0, gs1, gs2, ss0, ss1, ss2,
                    qs0, qs1, qs2, qd0, qd1, qd2):
    cid = lax.axis_index("c")
    sid = lax.axis_index("s")
    base = sid * RPT
    _stage_h(h_h.at[cid], h_sh, sid)
    pltpu.sync_copy(asrc_h.at[pl.ds(base, RPT)], asrc_sh.at[pl.ds(base, RPT)])
    pltpu.sync_copy(adst_h.at[pl.ds(base, RPT)], adst_sh.at[pl.ds(base, RPT)])
    _zero_rows(rows0)
    _zero_num_shared(rows0, num_sh, sid)
    _zero_vec(den_v)
    plsc.subcore_barrier()

    srcs = (src0, src1, src2)
    dsts = (dst0, dst1, dst2)
    es = (es0, es1, es2)
    ed = (ed0, ed1, ed2)
    qs = (qs0, qs1, qs2)
    qd = (qd0, qd1, qd2)

    def extra_gather(b):
        pltpu.async_copy(asrc_sh.at[srcs[b]], es[b], qs[b])
        pltpu.async_copy(adst_sh.at[dsts[b]], ed[b], qd[b])

    def extra_drain(b):
        pltpu.make_async_copy(asrc_sh.at[srcs[b]], es[b], qs[b]).wait()
        pltpu.make_async_copy(adst_sh.at[dsts[b]], ed[b], qd[b]).wait()

    def compute(src_v, dst_v, rows_v, b):
        extra_drain(b)
        es_v = es[b]
        ed_v = ed[b]

        def group(g, _):
            gbase = pl.multiple_of(g * 16, 16)
            d16 = dst_v[pl.ds(gbase, 16)]
            e = es_v[pl.ds(gbase, 16)] + ed_v[pl.ds(gbase, 16)]
            e = jnp.where(e >= 0.0, e, 0.2 * e)
            w = jnp.exp(e)
            plsc.addupdate_scatter(den_v, [d16], w)
            for l in range(16):
                ws = w[l]
                row = g * 16 + l
                for j in range(HC // 16):
                    sl = pl.ds(j * 16, 16)
                    rows_v[row, sl] = rows_v[row, sl] * ws
            return 0
        lax.fori_loop(0, KA // 16, group, 0)

    _edge_pipeline(src_h, dst_h, h_sh, num_h, num_sh,
                   srcs, dsts,
                   (rows0, rows1, rows2), (gs0, gs1, gs2), (ss0, ss1, ss2),
                   compute, cid, sid, KA, CPTA, extra_gather, extra_drain)

    @pl.when(cid == 0)
    def _():
        pltpu.sync_copy(den_v, den_h.at[sid])


def _gcn_edges_body(src_h, dst_h, h_h, num_h,
                    num_sh, h_sh, src0, src1, src2, dst0, dst1, dst2,
                    rows0, rows1, rows2, gs0, gs1, gs2, ss0, ss1, ss2):
    cid = lax.axis_index("c")
    sid = lax.axis_index("s")
    _stage_h(h_h.at[cid], h_sh, sid)
    _zero_rows(rows0)
    _zero_num_shared(rows0, num_sh, sid)
    plsc.subcore_barrier()
    _edge_pipeline(src_h, dst_h, h_sh, num_h, num_sh,
                   (src0, src1, src2), (dst0, dst1, dst2),
                   (rows0, rows1, rows2), (gs0, gs1, gs2), (ss0, ss1, ss2),
                   None, cid, sid, KG, CPTG)


def _degrees_body(dst_h, deg_h, den_v, dst_v):
    cid = lax.axis_index("c")
    sid = lax.axis_index("s")
    gw = cid * NS + sid
    _zero_vec(den_v)
    ones = jnp.ones((16,), jnp.float32)

    def chunk(c, _):
        base = (gw * CPD + c) * KD
        pltpu.sync_copy(dst_h.at[pl.ds(base, KD)], dst_v)
        for g in range(KD // 16):
            d16 = dst_v[pl.ds(g * 16, 16)]
            plsc.addupdate_scatter(den_v, [d16], ones)
        return 0
    lax.fori_loop(0, CPD, chunk, 0)
    pltpu.sync_copy(den_v, deg_h.at[gw])


@functools.lru_cache(maxsize=None)
def _sc_kernels():
    mesh = plsc.VectorSubcoreMesh(core_axis_name="c", subcore_axis_name="s",
                                  num_cores=NC, num_subcores=NS)
    cparams = pltpu.CompilerParams(needs_layout_passes=False,
                                   use_tc_tiling_on_sc=False)
    idx6a = [pltpu.VMEM((KA,), jnp.int32)] * 6
    rows3a = [pltpu.VMEM((KA, HC), jnp.float32)] * 3
    idx6g = [pltpu.VMEM((KG,), jnp.int32)] * 6
    rows3g = [pltpu.VMEM((KG, HC), jnp.float32)] * 3
    sems6 = [pltpu.SemaphoreType.DMA] * 6
    gat = pl.kernel(
        _gat_edges_body,
        out_type=(jax.ShapeDtypeStruct((NC, NP, HC), jnp.float32),
                  jax.ShapeDtypeStruct((NS, NP), jnp.float32)),
        mesh=mesh,
        compiler_params=cparams,
        scratch_types=[
            pltpu.VMEM_SHARED((NP, HC), jnp.float32),
            pltpu.VMEM_SHARED((NP, HC), jnp.float32),
            pltpu.VMEM_SHARED((NP,), jnp.float32),
            pltpu.VMEM_SHARED((NP,), jnp.float32),
            pltpu.VMEM((NP,), jnp.float32),
        ] + [pltpu.VMEM((KA,), jnp.float32)] * 6
          + idx6a + rows3a + sems6 + [pltpu.SemaphoreType.DMA] * 6,
    )
    gcn = pl.kernel(
        _gcn_edges_body,
        out_type=jax.ShapeDtypeStruct((NC, NP, HC), jnp.float32),
        mesh=mesh,
        compiler_params=cparams,
        scratch_types=[
            pltpu.VMEM_SHARED((NP, HC), jnp.float32),
            pltpu.VMEM_SHARED((NP, HC), jnp.float32),
        ] + idx6g + rows3g + sems6,
    )
    deg = pl.kernel(
        _degrees_body,
        out_type=jax.ShapeDtypeStruct((NW, NP), jnp.float32),
        mesh=mesh,
        compiler_params=cparams,
        scratch_types=[
            pltpu.VMEM((NP,), jnp.float32),
            pltpu.VMEM((KD,), jnp.int32),
        ],
    )
    return gat, gcn, deg


_BM = 256


def _split(h):
    return h[:, :HC], h[:, HC:]


def _mm(x, W):
    def body(x_ref, w_ref, o_ref):
        h = jnp.dot(x_ref[...], w_ref[...], preferred_element_type=jnp.float32)
        lo, hi = _split(h)
        o_ref[0] = lo
        o_ref[1] = hi
    return pl.pallas_call(
        body,
        grid=(NP // _BM,),
        in_specs=[pl.BlockSpec((_BM, D), lambda i: (i, 0)),
                  pl.BlockSpec((D, D), lambda i: (0, 0))],
        out_specs=pl.BlockSpec((NC, _BM, HC), lambda i: (0, i, 0)),
        out_shape=jax.ShapeDtypeStruct((NC, NP, HC), jnp.float32),
    )(x, W)


def _mm_scale(x, W, dinv):
    def body(x_ref, w_ref, di_ref, o_ref):
        h = di_ref[...] * jnp.dot(x_ref[...], w_ref[...],
                                  preferred_element_type=jnp.float32)
        lo, hi = _split(h)
        o_ref[0] = lo
        o_ref[1] = hi
    return pl.pallas_call(
        body,
        grid=(NP // _BM,),
        in_specs=[pl.BlockSpec((_BM, D), lambda i: (i, 0)),
                  pl.BlockSpec((D, D), lambda i: (0, 0)),
                  pl.BlockSpec((_BM, 1), lambda i: (i, 0))],
        out_specs=pl.BlockSpec((NC, _BM, HC), lambda i: (0, i, 0)),
        out_shape=jax.ShapeDtypeStruct((NC, NP, HC), jnp.float32),
    )(x, W, dinv)


def _mm_att(x, W, a2):
    def body(x_ref, w_ref, a_ref, h_ref, as_ref, ad_ref):
        h = jnp.dot(x_ref[...], w_ref[...], preferred_element_type=jnp.float32)
        av = jnp.dot(h, a_ref[...], preferred_element_type=jnp.float32)
        lo, hi = _split(h)
        h_ref[0] = lo
        h_ref[1] = hi
        as_ref[...] = av[:, 0:1]
        ad_ref[...] = av[:, 1:2]
    return pl.pallas_call(
        body,
        grid=(NP // _BM,),
        in_specs=[pl.BlockSpec((_BM, D), lambda i: (i, 0)),
                  pl.BlockSpec((D, D), lambda i: (0, 0)),
                  pl.BlockSpec((D, 2), lambda i: (0, 0))],
        out_specs=[pl.BlockSpec((NC, _BM, HC), lambda i: (0, i, 0)),
                   pl.BlockSpec((_BM, 1), lambda i: (i, 0)),
                   pl.BlockSpec((_BM, 1), lambda i: (i, 0))],
        out_shape=[jax.ShapeDtypeStruct((NC, NP, HC), jnp.float32),
                   jax.ShapeDtypeStruct((NP, 1), jnp.float32),
                   jax.ShapeDtypeStruct((NP, 1), jnp.float32)],
    )(x, W, a2)


def _cat(ref):
    return jnp.concatenate([ref[0], ref[1]], axis=-1)


def _gat_epilogue(num, den, h, asrc, adst, b, relu):
    def body(n_ref, d_ref, h_ref, as_ref, ad_ref, b_ref, o_ref):
        e = as_ref[...] + ad_ref[...]
        w_self = jnp.exp(jnp.where(e >= 0.0, e, 0.2 * e))
        den_tot = jnp.sum(d_ref[...], axis=0, keepdims=True).T + w_self + 1e-16
        numer = _cat(n_ref) + w_self * _cat(h_ref)
        out = numer / den_tot + b_ref[...]
        if relu:
            out = jnp.maximum(out, 0.0)
        o_ref[...] = out
    return pl.pallas_call(
        body,
        grid=(NP // _BM,),
        in_specs=[pl.BlockSpec((NC, _BM, HC), lambda i: (0, i, 0)),
                  pl.BlockSpec((NS, _BM), lambda i: (0, i)),
                  pl.BlockSpec((NC, _BM, HC), lambda i: (0, i, 0)),
                  pl.BlockSpec((_BM, 1), lambda i: (i, 0)),
                  pl.BlockSpec((_BM, 1), lambda i: (i, 0)),
                  pl.BlockSpec((1, D), lambda i: (0, 0))],
        out_specs=pl.BlockSpec((_BM, D), lambda i: (i, 0)),
        out_shape=jax.ShapeDtypeStruct((NP, D), jnp.float32),
    )(num, den, h, asrc, adst, b.reshape(1, D))


def _gcn_epilogue(num, hp, dinv, b, relu):
    def body(n_ref, h_ref, di_ref, b_ref, o_ref):
        out = di_ref[...] * (_cat(n_ref) + _cat(h_ref)) + b_ref[...]
        if relu:
            out = jnp.maximum(out, 0.0)
        o_ref[...] = out
    return pl.pallas_call(
        body,
        grid=(NP // _BM,),
        in_specs=[pl.BlockSpec((NC, _BM, HC), lambda i: (0, i, 0)),
                  pl.BlockSpec((NC, _BM, HC), lambda i: (0, i, 0)),
                  pl.BlockSpec((_BM, 1), lambda i: (i, 0)),
                  pl.BlockSpec((1, D), lambda i: (0, 0))],
        out_specs=pl.BlockSpec((_BM, D), lambda i: (i, 0)),
        out_shape=jax.ShapeDtypeStruct((NP, D), jnp.float32),
    )(num, hp, dinv, b.reshape(1, D))


def _dinv_kernel(deg):
    def body(deg_ref, o_ref):
        tot = jnp.sum(deg_ref[...], axis=0, keepdims=True) + 1.0
        o_ref[...] = lax.rsqrt(tot)
    return pl.pallas_call(
        body,
        out_shape=jax.ShapeDtypeStruct((1, NP), jnp.float32),
    )(deg)


def _fusion(e1, e2, c1, c2, aW1, ab1, aW2):
    def body(e1_ref, e2_ref, c1_ref, c2_ref, w1_ref, b1_ref, w2_ref, o_ref):
        z0 = e1_ref[...]
        z1 = e2_ref[...]
        z2 = (c1_ref[...] + c2_ref[...]) * 0.5

        def att(z):
            t = jnp.tanh(jnp.dot(z, w1_ref[...],
                                 preferred_element_type=jnp.float32)
                         + b1_ref[...])
            return jnp.dot(t, w2_ref[...], preferred_element_type=jnp.float32)

        w0, w1, w2 = att(z0), att(z1), att(z2)
        m = jnp.maximum(jnp.maximum(w0, w1), w2)
        x0 = jnp.exp(w0 - m)
        x1 = jnp.exp(w1 - m)
        x2 = jnp.exp(w2 - m)
        s = x0 + x1 + x2
        o_ref[...] = (x0 * z0 + x1 * z1 + x2 * z2) / s
    return pl.pallas_call(
        body,
        grid=(NP // _BM,),
        in_specs=[pl.BlockSpec((_BM, D), lambda i: (i, 0))] * 4
                 + [pl.BlockSpec((D, 16), lambda i: (0, 0)),
                    pl.BlockSpec((1, 16), lambda i: (0, 0)),
                    pl.BlockSpec((16, 1), lambda i: (0, 0))],
        out_specs=pl.BlockSpec((_BM, D), lambda i: (i, 0)),
        out_shape=jax.ShapeDtypeStruct((NP, D), jnp.float32),
    )(e1, e2, c1, c2, aW1, ab1, aW2)


def _gat_layer(x, W, a_s, a_d, b, src, dst, relu):
    gat_edges, _, _ = _sc_kernels()
    h, asrc, adst = _mm_att(x, W, jnp.stack([a_s, a_d], axis=1))
    num, den = gat_edges(src, dst, h, asrc.reshape(NP), adst.reshape(NP))
    return _gat_epilogue(num, den, h, asrc, adst, b, relu)


def _gcn_layer(x, W, b, src, dst, dinv_col, relu):
    _, gcn_edges, _ = _sc_kernels()
    hp = _mm_scale(x, W, dinv_col)
    num = gcn_edges(src, dst, hp)
    return _gcn_epilogue(num, hp, dinv_col, b, relu)


def kernel(adj, aug_feat1, aug_feat2,
           g1_W1, g1_as1, g1_ad1, g1_b1, g1_W2, g1_as2, g1_ad2, g1_b2,
           g2_W1, g2_as1, g2_ad1, g2_b1, g2_W2, g2_as2, g2_ad2, g2_b2,
           c_W1, c_b1, c_W2, c_b2, a_W1, a_b1, a_W2):
    pad_e = jnp.full((EPAD - E,), N, jnp.int32)
    src = jnp.concatenate([adj[0], pad_e])
    dst = jnp.concatenate([adj[1], pad_e])
    x1 = jnp.pad(aug_feat1, ((0, NP - N), (0, 0)))
    x2 = jnp.pad(aug_feat2, ((0, NP - N), (0, 0)))

    _, _, degrees = _sc_kernels()
    deg = degrees(dst)
    dinv_col = _dinv_kernel(deg).reshape(NP, 1)

    h1 = _gat_layer(x1, g1_W1, g1_as1, g1_ad1, g1_b1, src, dst, True)
    emb1 = _gat_layer(h1, g1_W2, g1_as2, g1_ad2, g1_b2, src, dst, False)
    h2 = _gat_layer(x2, g2_W1, g2_as1, g2_ad1, g2_b1, src, dst, True)
    emb2 = _gat_layer(h2, g2_W2, g2_as2, g2_ad2, g2_b2, src, dst, False)

    hc1 = _gcn_layer(x1, c_W1, c_b1, src, dst, dinv_col, True)
    com1 = _gcn_layer(hc1, c_W2, c_b2, src, dst, dinv_col, False)
    hc2 = _gcn_layer(x2, c_W1, c_b1, src, dst, dinv_col, True)
    com2 = _gcn_layer(hc2, c_W2, c_b2, src, dst, dinv_col, False)

    out = _fusion(emb1, emb2, com1, com2, a_W1, a_b1.reshape(1, 16), a_W2)
    return out[:N]
